# baseline (speedup 1.0000x reference)
import functools

import jax
import jax.numpy as jnp
from jax import lax
from jax.experimental import pallas as pl
from jax.experimental.pallas import tpu as pltpu
from jax.experimental.pallas import tpu_sc as plsc

B = 16384
D = 64
NUM_ITEM_HALF = 500000
NC = 2
NS = 16
NW = NC * NS
BPW = B // NW
CHUNK = 128
NCHUNK = BPW // CHUNK
GROUPS = BPW // 16


def _gmf_kernel(utab_hbm, itlo_hbm, ithi_hbm, uid_hbm, iidlo_hbm, iidhi_hbm,
                iidsel_hbm, w_hbm, b_hbm, out_hbm,
                uidx, ilox, ihix, isel, urows, vlo, vhi, wvec, bvec, outv,
                sem):
    wid = lax.axis_index("s") * NC + lax.axis_index("c")

    pltpu.sync_copy(uid_hbm.at[pl.ds(wid * NCHUNK, NCHUNK)], uidx)
    pltpu.sync_copy(iidlo_hbm.at[pl.ds(wid * NCHUNK, NCHUNK)], ilox)
    pltpu.sync_copy(iidhi_hbm.at[pl.ds(wid * NCHUNK, NCHUNK)], ihix)
    pltpu.sync_copy(iidsel_hbm.at[pl.ds(wid * BPW, BPW)], isel)
    pltpu.sync_copy(w_hbm, wvec)
    pltpu.sync_copy(b_hbm, bvec)

    copies = []
    for j in range(NCHUNK):
        sl = pl.ds(j * CHUNK, CHUNK)
        copies.append(pltpu.async_copy(
            utab_hbm.at[uidx.at[j]], urows.at[sl], sem))
        copies.append(pltpu.async_copy(
            itlo_hbm.at[ilox.at[j]], vlo.at[sl], sem))
        copies.append(pltpu.async_copy(
            ithi_hbm.at[ihix.at[j]], vhi.at[sl], sem))
    for c in copies:
        c.wait()

    iota = lax.broadcasted_iota(jnp.int32, (16,), 0)
    bias = bvec[...]
    bscal = bias[0]
    wvals = [wvec[pl.ds(j * 16, 16)] for j in range(D // 16)]

    def group_body(g, carry):
        sels = isel[pl.ds(g * 16, 16)]
        acc = bias
        for r in range(16):
            row = g * 16 + r
            in_lo = sels[r] < NUM_ITEM_HALF
            t = None
            for j in range(D // 16):
                sl = pl.ds(j * 16, 16)
                v = jnp.where(in_lo, vlo[row, sl], vhi[row, sl])
                term = (urows[row, sl] * v) * wvals[j]
                t = term if t is None else t + term
            s = jnp.sum(t) + bscal
            acc = jnp.where(iota == r, s, acc)
        outv[pl.ds(g * 16, 16)] = acc
        return carry

    lax.fori_loop(0, GROUPS, group_body, 0)

    pltpu.sync_copy(outv, out_hbm.at[pl.ds(wid * BPW, BPW)])


def kernel(user_id, item_id, user_table, item_table, linear_w, linear_b):
    uid2d = user_id.reshape(NW * NCHUNK, CHUNK).astype(jnp.int32)
    iid = item_id.astype(jnp.int32)
    iidlo2d = jnp.minimum(iid, NUM_ITEM_HALF - 1).reshape(NW * NCHUNK, CHUNK)
    iidhi2d = jnp.maximum(iid - NUM_ITEM_HALF, 0).reshape(NW * NCHUNK, CHUNK)
    it_lo = item_table[:NUM_ITEM_HALF]
    it_hi = item_table[NUM_ITEM_HALF:]
    w = linear_w.reshape(D)
    b16 = jnp.broadcast_to(linear_b.reshape(()), (16,)).astype(jnp.float32)

    run = functools.partial(
        pl.kernel,
        mesh=plsc.VectorSubcoreMesh(core_axis_name="c", subcore_axis_name="s"),
        out_type=jax.ShapeDtypeStruct((B,), jnp.float32),
        compiler_params=pltpu.CompilerParams(
            needs_layout_passes=False, use_tc_tiling_on_sc=False,
            skip_device_barrier=True),
        scratch_types=[
            pltpu.VMEM((NCHUNK, CHUNK), jnp.int32),
            pltpu.VMEM((NCHUNK, CHUNK), jnp.int32),
            pltpu.VMEM((NCHUNK, CHUNK), jnp.int32),
            pltpu.VMEM((BPW,), jnp.int32),
            pltpu.VMEM((BPW, D), jnp.float32),
            pltpu.VMEM((BPW, D), jnp.float32),
            pltpu.VMEM((BPW, D), jnp.float32),
            pltpu.VMEM((D,), jnp.float32),
            pltpu.VMEM((16,), jnp.float32),
            pltpu.VMEM((BPW,), jnp.float32),
            pltpu.SemaphoreType.DMA,
        ],
    )(_gmf_kernel)

    return run(user_table, it_lo, it_hi, uid2d, iidlo2d, iidhi2d, iid, w, b16)


# speedup vs baseline: 1.7086x; 1.7086x over previous
import functools

import jax
import jax.numpy as jnp
from jax import lax
from jax.experimental import pallas as pl
from jax.experimental.pallas import tpu as pltpu
from jax.experimental.pallas import tpu_sc as plsc

B = 16384
D = 64
NW = 16
BPW = B // NW
CHUNK = 256
NCHUNK = BPW // CHUNK
CGROUPS = CHUNK // 16


def _gmf_kernel(utab_hbm, itab_hbm, uid_hbm, iid_hbm, w_hbm, b_hbm, out_hbm,
                uidx, iidx, ubuf, vbuf, wvec, bvec, outv, sem0, sem1):
    wid = lax.axis_index("s")
    sems = (sem0, sem1)

    pltpu.sync_copy(uid_hbm.at[pl.ds(wid * (BPW // 128), BPW // 128)], uidx)
    pltpu.sync_copy(iid_hbm.at[pl.ds(wid * (BPW // 128), BPW // 128)], iidx)
    pltpu.sync_copy(w_hbm, wvec)
    pltpu.sync_copy(b_hbm, bvec)

    def issue_chunk(k, p):
        for h in range(2):
            row = pl.ds(h * 128, 128)
            pltpu.async_copy(
                utab_hbm.at[uidx.at[k * 2 + h]], ubuf.at[p, row], sems[p])
            pltpu.async_copy(
                itab_hbm.at[iidx.at[k * 2 + h]], vbuf.at[p, row], sems[p])

    def drain_chunk(p):
        pltpu.make_async_copy(
            utab_hbm.at[pl.ds(0, CHUNK)], ubuf.at[p], sems[p]).wait()
        pltpu.make_async_copy(
            itab_hbm.at[pl.ds(0, CHUNK)], vbuf.at[p], sems[p]).wait()

    iota = lax.broadcasted_iota(jnp.int32, (16,), 0)
    bias = bvec[...]
    bscal = bias[0]
    wvals = [wvec[pl.ds(j * 16, 16)] for j in range(D // 16)]

    def compute_chunk(k, p):
        def group_body(g, carry):
            acc = bias
            for r in range(16):
                row = g * 16 + r
                t = (ubuf[p, row, pl.ds(0, 16)]
                     * vbuf[p, row, pl.ds(0, 16)]) * wvals[0]
                for j in range(1, D // 16):
                    t = t + (ubuf[p, row, pl.ds(j * 16, 16)]
                             * vbuf[p, row, pl.ds(j * 16, 16)]) * wvals[j]
                s = jnp.sum(t) + bscal
                acc = jnp.where(iota == r, s, acc)
            outv[pl.ds(k * CHUNK + g * 16, 16)] = acc
            return carry
        lax.fori_loop(0, CGROUPS, group_body, 0)

    issue_chunk(0, 0)
    for k in range(NCHUNK):
        p = k % 2
        if k + 1 < NCHUNK:
            issue_chunk(k + 1, 1 - p)
        drain_chunk(p)
        compute_chunk(k, p)

    pltpu.sync_copy(outv, out_hbm.at[pl.ds(wid * BPW, BPW)])


def kernel(user_id, item_id, user_table, item_table, linear_w, linear_b):
    uid2d = user_id.reshape(B // 128, 128).astype(jnp.int32)
    iid2d = item_id.reshape(B // 128, 128).astype(jnp.int32)
    w = linear_w.reshape(D)
    b16 = jnp.broadcast_to(linear_b.reshape(()), (16,)).astype(jnp.float32)

    run = functools.partial(
        pl.kernel,
        mesh=plsc.VectorSubcoreMesh(
            core_axis_name="c", subcore_axis_name="s", num_cores=1),
        out_type=jax.ShapeDtypeStruct((B,), jnp.float32),
        compiler_params=pltpu.CompilerParams(
            needs_layout_passes=False, use_tc_tiling_on_sc=False,
            skip_device_barrier=True),
        scratch_types=[
            pltpu.VMEM((BPW // 128, 128), jnp.int32),
            pltpu.VMEM((BPW // 128, 128), jnp.int32),
            pltpu.VMEM((2, CHUNK, D), jnp.float32),
            pltpu.VMEM((2, CHUNK, D), jnp.float32),
            pltpu.VMEM((D,), jnp.float32),
            pltpu.VMEM((16,), jnp.float32),
            pltpu.VMEM((BPW,), jnp.float32),
            pltpu.SemaphoreType.DMA,
            pltpu.SemaphoreType.DMA,
        ],
    )(_gmf_kernel)

    return run(user_table, item_table, uid2d, iid2d, w, b16)


# speedup vs baseline: 2.5749x; 1.5070x over previous
import functools

import jax
import jax.numpy as jnp
from jax import lax
from jax.experimental import pallas as pl
from jax.experimental.pallas import tpu as pltpu
from jax.experimental.pallas import tpu_sc as plsc

B = 16384
D = 64
NC = 2
NS = 16
NW = NC * NS
BPW = B // NW
GCHUNK = 128
NGCHUNK = BPW // GCHUNK
CHUNK = 128
NCHUNK = BPW // CHUNK
CGROUPS = CHUNK // 16


def _user_gather_kernel(utab_hbm, uid_hbm, urows_hbm, uidx, ubuf, sem):
    wid = lax.axis_index("s") * NC + lax.axis_index("c")
    pltpu.sync_copy(uid_hbm.at[pl.ds(wid * NGCHUNK, NGCHUNK)], uidx)
    copies = []
    for j in range(NGCHUNK):
        copies.append(pltpu.async_copy(
            utab_hbm.at[uidx.at[j]], ubuf.at[pl.ds(j * GCHUNK, GCHUNK)], sem))
    for c in copies:
        c.wait()
    pltpu.sync_copy(ubuf, urows_hbm.at[pl.ds(wid * BPW, BPW)])


def _item_compute_kernel(itab_hbm, urows_hbm, iid_hbm, w_hbm, b_hbm, out_hbm,
                         iids, ustage, vbuf, wvec, bvec, outv, sem0, sem1):
    wid = lax.axis_index("s") * NC + lax.axis_index("c")
    sems = (sem0, sem1)

    pltpu.sync_copy(iid_hbm.at[pl.ds(wid * BPW, BPW)], iids)
    pltpu.sync_copy(w_hbm, wvec)
    pltpu.sync_copy(b_hbm, bvec)
    pltpu.sync_copy(urows_hbm.at[pl.ds(wid * BPW, BPW)], ustage)

    def issue_chunk(k, p):
        def body(g, carry):
            ivec = iids[pl.ds(k * CHUNK + g * 16, 16)]
            for l in range(16):
                row = g * 16 + l
                pltpu.async_copy(itab_hbm.at[ivec[l]], vbuf.at[p, row], sems[p])
            return carry
        lax.fori_loop(0, CGROUPS, body, 0)

    def drain_chunk(p):
        pltpu.make_async_copy(
            itab_hbm.at[pl.ds(0, CHUNK)], vbuf.at[p], sems[p]).wait()

    iota = lax.broadcasted_iota(jnp.int32, (16,), 0)
    bias = bvec[...]
    bscal = bias[0]
    wvals = [wvec[pl.ds(j * 16, 16)] for j in range(D // 16)]

    def compute_chunk(k, p):
        def group_body(g, carry):
            acc = bias
            for r in range(16):
                row = g * 16 + r
                grow = k * CHUNK + row
                t = (ustage[grow, pl.ds(0, 16)]
                     * vbuf[p, row, pl.ds(0, 16)]) * wvals[0]
                for j in range(1, D // 16):
                    t = t + (ustage[grow, pl.ds(j * 16, 16)]
                             * vbuf[p, row, pl.ds(j * 16, 16)]) * wvals[j]
                s = jnp.sum(t) + bscal
                acc = jnp.where(iota == r, s, acc)
            outv[pl.ds(k * CHUNK + g * 16, 16)] = acc
            return carry
        lax.fori_loop(0, CGROUPS, group_body, 0)

    issue_chunk(0, 0)
    for k in range(NCHUNK):
        p = k % 2
        if k + 1 < NCHUNK:
            issue_chunk(k + 1, 1 - p)
        drain_chunk(p)
        compute_chunk(k, p)

    pltpu.sync_copy(outv, out_hbm.at[pl.ds(wid * BPW, BPW)])


def kernel(user_id, item_id, user_table, item_table, linear_w, linear_b):
    uid2d = user_id.reshape(NW * NGCHUNK, GCHUNK).astype(jnp.int32)
    iid = item_id.astype(jnp.int32)
    w = linear_w.reshape(D)
    b16 = jnp.broadcast_to(linear_b.reshape(()), (16,)).astype(jnp.float32)

    mesh = plsc.VectorSubcoreMesh(core_axis_name="c", subcore_axis_name="s")

    gather_u = functools.partial(
        pl.kernel,
        mesh=mesh,
        out_type=jax.ShapeDtypeStruct((B, D), jnp.float32),
        compiler_params=pltpu.CompilerParams(
            needs_layout_passes=False, use_tc_tiling_on_sc=False),
        scratch_types=[
            pltpu.VMEM((NGCHUNK, GCHUNK), jnp.int32),
            pltpu.VMEM((BPW, D), jnp.float32),
            pltpu.SemaphoreType.DMA,
        ],
    )(_user_gather_kernel)

    item_compute = functools.partial(
        pl.kernel,
        mesh=mesh,
        out_type=jax.ShapeDtypeStruct((B,), jnp.float32),
        compiler_params=pltpu.CompilerParams(needs_layout_passes=False),
        scratch_types=[
            pltpu.VMEM((BPW,), jnp.int32),
            pltpu.VMEM((BPW, D), jnp.float32),
            pltpu.VMEM((2, CHUNK, D), jnp.float32),
            pltpu.VMEM((D,), jnp.float32),
            pltpu.VMEM((16,), jnp.float32),
            pltpu.VMEM((BPW,), jnp.float32),
            pltpu.SemaphoreType.DMA,
            pltpu.SemaphoreType.DMA,
        ],
    )(_item_compute_kernel)

    urows = gather_u(user_table, uid2d)
    return item_compute(item_table, urows, iid, w, b16)


# speedup vs baseline: 2.8091x; 1.0910x over previous
import functools

import jax
import jax.numpy as jnp
from jax import lax
from jax.experimental import pallas as pl
from jax.experimental.pallas import tpu as pltpu
from jax.experimental.pallas import tpu_sc as plsc

B = 16384
D = 64
NC = 2
NS = 16
NW = NC * NS
BPW = B // NW
CHUNK = 128
NCHUNK = BPW // CHUNK
CGROUPS = CHUNK // 16


def _gmf_kernel(utab_hbm, itab_hbm, uid_hbm, iid_hbm, w_hbm, b_hbm, out_hbm,
                uids, iids, ubuf, vbuf, wvec, bvec, outv, sem0, sem1):
    wid = lax.axis_index("s") * NC + lax.axis_index("c")
    sems = (sem0, sem1)

    pltpu.sync_copy(uid_hbm.at[pl.ds(wid * BPW, BPW)], uids)
    pltpu.sync_copy(iid_hbm.at[pl.ds(wid * BPW, BPW)], iids)
    pltpu.sync_copy(w_hbm, wvec)
    pltpu.sync_copy(b_hbm, bvec)

    def issue_chunk(k, p):
        def body(g, carry):
            uvec = uids[pl.ds(k * CHUNK + g * 16, 16)]
            ivec = iids[pl.ds(k * CHUNK + g * 16, 16)]
            for l in range(16):
                row = g * 16 + l
                pltpu.async_copy(utab_hbm.at[uvec[l]], ubuf.at[p, row], sems[p])
                pltpu.async_copy(itab_hbm.at[ivec[l]], vbuf.at[p, row], sems[p])
            return carry
        lax.fori_loop(0, CGROUPS, body, 0)

    def drain_chunk(p):
        pltpu.make_async_copy(
            utab_hbm.at[pl.ds(0, CHUNK)], ubuf.at[p], sems[p]).wait()
        pltpu.make_async_copy(
            itab_hbm.at[pl.ds(0, CHUNK)], vbuf.at[p], sems[p]).wait()

    iota = lax.broadcasted_iota(jnp.int32, (16,), 0)
    bias = bvec[...]
    bscal = bias[0]
    wvals = [wvec[pl.ds(j * 16, 16)] for j in range(D // 16)]

    def compute_chunk(k, p):
        def group_body(g, carry):
            acc = bias
            for r in range(16):
                row = g * 16 + r
                t = (ubuf[p, row, pl.ds(0, 16)]
                     * vbuf[p, row, pl.ds(0, 16)]) * wvals[0]
                for j in range(1, D // 16):
                    t = t + (ubuf[p, row, pl.ds(j * 16, 16)]
                             * vbuf[p, row, pl.ds(j * 16, 16)]) * wvals[j]
                s = jnp.sum(t) + bscal
                acc = jnp.where(iota == r, s, acc)
            outv[pl.ds(k * CHUNK + g * 16, 16)] = acc
            return carry
        lax.fori_loop(0, CGROUPS, group_body, 0)

    issue_chunk(0, 0)
    for k in range(NCHUNK):
        p = k % 2
        if k + 1 < NCHUNK:
            issue_chunk(k + 1, 1 - p)
        drain_chunk(p)
        compute_chunk(k, p)

    pltpu.sync_copy(outv, out_hbm.at[pl.ds(wid * BPW, BPW)])


def kernel(user_id, item_id, user_table, item_table, linear_w, linear_b):
    uid = user_id.astype(jnp.int32)
    iid = item_id.astype(jnp.int32)
    w = linear_w.reshape(D)
    b16 = jnp.broadcast_to(linear_b.reshape(()), (16,)).astype(jnp.float32)

    run = functools.partial(
        pl.kernel,
        mesh=plsc.VectorSubcoreMesh(core_axis_name="c", subcore_axis_name="s"),
        out_type=jax.ShapeDtypeStruct((B,), jnp.float32),
        compiler_params=pltpu.CompilerParams(needs_layout_passes=False),
        scratch_types=[
            pltpu.VMEM((BPW,), jnp.int32),
            pltpu.VMEM((BPW,), jnp.int32),
            pltpu.VMEM((2, CHUNK, D), jnp.float32),
            pltpu.VMEM((2, CHUNK, D), jnp.float32),
            pltpu.VMEM((D,), jnp.float32),
            pltpu.VMEM((16,), jnp.float32),
            pltpu.VMEM((BPW,), jnp.float32),
            pltpu.SemaphoreType.DMA,
            pltpu.SemaphoreType.DMA,
        ],
    )(_gmf_kernel)

    return run(user_table, item_table, uid, iid, w, b16)
